# baseline (speedup 1.0000x reference)
import functools

import jax
import jax.numpy as jnp
from jax import lax
from jax.experimental import pallas as pl
from jax.experimental.pallas import tpu as pltpu
from jax.experimental.pallas import tpu_sc as plsc

D_MODEL = 64
SCALE = 8.0
NC, NS = 2, 16
NW = NC * NS
XROWS = 4096
SEQ = 200
RPW = XROWS // NW
SPLIT = 128
LANES = 16
NBUF = 4
AHEAD = 2


def _body(x_hbm, table_hbm, out_hbm, idx_v, *rest):
    bufs = rest[:NBUF]
    sgs = rest[NBUF:2 * NBUF]
    sos = rest[2 * NBUF:3 * NBUF]
    c = lax.axis_index("c")
    s = lax.axis_index("s")
    wid = s * NC + c
    base = wid * RPW
    pltpu.sync_copy(x_hbm.at[pl.ds(base, RPW)], idx_v)

    def gather_parts(r, b):
        return (
            (table_hbm.at[idx_v.at[r, pl.ds(0, SPLIT)]],
             bufs[b].at[pl.ds(0, SPLIT)]),
            (table_hbm.at[idx_v.at[r, pl.ds(SPLIT, SEQ - SPLIT)]],
             bufs[b].at[pl.ds(SPLIT, SEQ - SPLIT)]),
        )

    def issue_gather(r, b):
        for src, dst in gather_parts(r, b):
            pltpu.async_copy(src, dst, sgs[b])

    def wait_gather(r, b):
        for src, dst in gather_parts(r, b):
            pltpu.make_async_copy(src, dst, sgs[b]).wait()

    def issue_out(r, b):
        pltpu.async_copy(bufs[b], out_hbm.at[base + r], sos[b])

    def wait_out(r, b):
        pltpu.make_async_copy(bufs[b], out_hbm.at[base + r], sos[b]).wait()

    def scale(b):
        buf = bufs[b]

        def row4(i, _):
            q = i * 4
            for v in range(4):
                for u in range(D_MODEL // LANES):
                    sl = pl.ds(u * LANES, LANES)
                    buf[q + v, sl] = buf[q + v, sl] * SCALE
            return 0

        lax.fori_loop(0, SEQ // 4, row4, 0)

    def step(r, b, first=False):
        wait_gather(r, b)
        scale(b)
        issue_out(r, b)
        rn = r + AHEAD
        bn = (b + AHEAD) % NBUF
        if not first:
            wait_out(rn - NBUF, bn)
        issue_gather(rn, bn)

    for r in range(AHEAD):
        issue_gather(r, r)
    for b in range(NBUF):
        step(b, b, first=(b < AHEAD))

    def block(g, _):
        r0 = g * NBUF
        for b in range(NBUF):
            step(r0 + b, b)
        return 0

    lax.fori_loop(1, RPW // NBUF - 1, block, 0)

    r0 = RPW - NBUF
    for b in range(AHEAD):
        step(r0 + b, b)
    for b in range(AHEAD, NBUF):
        r = r0 + b
        wait_gather(r, b)
        scale(b)
        issue_out(r, b)
    for b in range(NBUF):
        wait_out(r0 + b, b)


_sc_call = functools.partial(
    pl.kernel,
    out_type=jax.ShapeDtypeStruct((XROWS, SEQ, D_MODEL), jnp.float32),
    mesh=plsc.VectorSubcoreMesh(core_axis_name="c", subcore_axis_name="s"),
    compiler_params=pltpu.CompilerParams(use_tc_tiling_on_sc=False),
    scratch_types=(
        [pltpu.VMEM((RPW, SEQ), jnp.int32)]
        + [pltpu.VMEM((SEQ, D_MODEL), jnp.float32) for _ in range(NBUF)]
        + [pltpu.SemaphoreType.DMA for _ in range(2 * NBUF)]
    ),
)(_body)


@jax.jit
def kernel(x, table):
    return _sc_call(x, table)
